# Initial kernel scaffold; baseline (speedup 1.0000x reference)
import jax
import jax.numpy as jnp
from jax.experimental import pallas as pl


def kernel(inputs, embeddings):
    raise NotImplementedError("write your pallas kernel here")


# speedup vs baseline: 1.1132x; 1.1132x over previous
import functools

import jax
import jax.numpy as jnp
from jax import lax
from jax.experimental import pallas as pl
from jax.experimental.pallas import tpu as pltpu
from jax.experimental.pallas import tpu_sc as plsc

NC, NS = 2, 16
NW = NC * NS
ROWS = 16384 * 50
D = 32
IDX_COLS = 128
ROWS_PER_W = ROWS // NW
IROWS_PER_W = ROWS_PER_W // IDX_COLS
K = 5
CHUNK = K * IDX_COLS
NCHUNKS = IROWS_PER_W // K
NBUF = 4
G = 2

_mesh = plsc.VectorSubcoreMesh(
    core_axis_name="c", subcore_axis_name="s", num_cores=NC, num_subcores=NS
)


@functools.partial(
    pl.kernel,
    out_type=jax.ShapeDtypeStruct((ROWS, D), jnp.float32),
    mesh=_mesh,
    scratch_types=[
        pltpu.VMEM((IROWS_PER_W, IDX_COLS), jnp.int32),
        pltpu.VMEM((NBUF, CHUNK, D), jnp.float32),
        pltpu.SemaphoreType.DMA((NBUF,)),
        pltpu.SemaphoreType.DMA((NBUF,)),
    ],
    compiler_params=pltpu.CompilerParams(use_tc_tiling_on_sc=False),
)
def _gather_kernel(table_hbm, idx_hbm, out_hbm, idx_v, rows_v, gsem, osem):
    wid = lax.axis_index("s") * NC + lax.axis_index("c")
    ibase = wid * IROWS_PER_W
    obase = wid * ROWS_PER_W
    pltpu.sync_copy(idx_hbm.at[pl.ds(ibase, IROWS_PER_W)], idx_v)

    def fire(c, b):
        for j in range(K):
            pltpu.async_copy(
                table_hbm.at[idx_v.at[c * K + j]],
                rows_v.at[b, pl.ds(j * IDX_COLS, IDX_COLS)],
                gsem.at[b],
            )

    def gwait(b):
        pltpu.make_async_copy(
            table_hbm.at[pl.ds(0, CHUNK)], rows_v.at[b], gsem.at[b]
        ).wait()

    def ostart(c, b):
        pltpu.async_copy(
            rows_v.at[b], out_hbm.at[pl.ds(obase + c * CHUNK, CHUNK)], osem.at[b]
        )

    def owait(b):
        pltpu.make_async_copy(
            rows_v.at[b], out_hbm.at[pl.ds(obase, CHUNK)], osem.at[b]
        ).wait()

    for b in range(G):
        fire(b, b)

    def body(c0):
        for b in range(NBUF):
            c = c0 + b
            bf = (b + G) % NBUF

            @pl.when(c + G < NCHUNKS)
            def _():
                @pl.when(c + G >= NBUF)
                def _():
                    owait(bf)

                fire(c + G, bf)

            gwait(b)
            ostart(c, b)

    pl.loop(0, NCHUNKS, step=NBUF)(body)

    for b in range(NBUF):
        owait(b)


@jax.jit
def kernel(inputs, embeddings):
    idx = inputs.reshape(ROWS // IDX_COLS, IDX_COLS).astype(jnp.int32)
    out = _gather_kernel(embeddings, idx)
    return out.reshape(inputs.shape + (D,))


# speedup vs baseline: 1.1137x; 1.0004x over previous
import functools

import jax
import jax.numpy as jnp
from jax import lax
from jax.experimental import pallas as pl
from jax.experimental.pallas import tpu as pltpu
from jax.experimental.pallas import tpu_sc as plsc

NC, NS = 2, 16
NW = NC * NS
ROWS = 16384 * 50
D = 32
IDX_COLS = 128
ROWS_PER_W = ROWS // NW
IROWS_PER_W = ROWS_PER_W // IDX_COLS
K = 2
CHUNK = K * IDX_COLS
NCHUNKS = IROWS_PER_W // K
NBUF = 10
G = 6

_mesh = plsc.VectorSubcoreMesh(
    core_axis_name="c", subcore_axis_name="s", num_cores=NC, num_subcores=NS
)


@functools.partial(
    pl.kernel,
    out_type=jax.ShapeDtypeStruct((ROWS, D), jnp.float32),
    mesh=_mesh,
    scratch_types=[
        pltpu.VMEM((IROWS_PER_W, IDX_COLS), jnp.int32),
        pltpu.VMEM((NBUF, CHUNK, D), jnp.float32),
        pltpu.SemaphoreType.DMA((NBUF,)),
        pltpu.SemaphoreType.DMA((NBUF,)),
    ],
    compiler_params=pltpu.CompilerParams(use_tc_tiling_on_sc=False),
)
def _gather_kernel(table_hbm, idx_hbm, out_hbm, idx_v, rows_v, gsem, osem):
    wid = lax.axis_index("s") * NC + lax.axis_index("c")
    ibase = wid * IROWS_PER_W
    obase = wid * ROWS_PER_W
    pltpu.sync_copy(idx_hbm.at[pl.ds(ibase, IROWS_PER_W)], idx_v)

    def fire(c, b):
        for j in range(K):
            pltpu.async_copy(
                table_hbm.at[idx_v.at[c * K + j]],
                rows_v.at[b, pl.ds(j * IDX_COLS, IDX_COLS)],
                gsem.at[b],
            )

    def gwait(b):
        pltpu.make_async_copy(
            table_hbm.at[pl.ds(0, CHUNK)], rows_v.at[b], gsem.at[b]
        ).wait()

    def ostart(c, b):
        pltpu.async_copy(
            rows_v.at[b], out_hbm.at[pl.ds(obase + c * CHUNK, CHUNK)], osem.at[b]
        )

    def owait(b):
        pltpu.make_async_copy(
            rows_v.at[b], out_hbm.at[pl.ds(obase, CHUNK)], osem.at[b]
        ).wait()

    for b in range(G):
        fire(b, b)

    def body(c0):
        for b in range(NBUF):
            c = c0 + b
            bf = (b + G) % NBUF

            @pl.when(c + G < NCHUNKS)
            def _():
                @pl.when(c + G >= NBUF)
                def _():
                    owait(bf)

                fire(c + G, bf)

            gwait(b)
            ostart(c, b)

    pl.loop(0, NCHUNKS, step=NBUF)(body)

    for b in range(NBUF):
        owait(b)


@jax.jit
def kernel(inputs, embeddings):
    idx = inputs.reshape(ROWS // IDX_COLS, IDX_COLS).astype(jnp.int32)
    out = _gather_kernel(embeddings, idx)
    return out.reshape(inputs.shape + (D,))


# speedup vs baseline: 2.4079x; 2.1621x over previous
import functools

import jax
import jax.numpy as jnp
from jax import lax
from jax.experimental import pallas as pl
from jax.experimental.pallas import tpu as pltpu
from jax.experimental.pallas import tpu_sc as plsc

NC, NS = 2, 16
NW = NC * NS
B = 16384
S = 50
D = 32
L = 128
NITEMS = S * (B // L)
IPW = NITEMS // NW
FT = D // 8
NG = 4
NO = 4

_mesh = plsc.VectorSubcoreMesh(
    core_axis_name="c", subcore_axis_name="s", num_cores=NC, num_subcores=NS
)


@functools.partial(
    pl.kernel,
    out_type=jax.ShapeDtypeStruct((S, FT, B // L, 8, L), jnp.float32),
    mesh=_mesh,
    scratch_types=[
        pltpu.VMEM((IPW, L), jnp.int32),
        pltpu.VMEM((NG, L, D), jnp.float32),
        pltpu.VMEM((NO, D, L + 9), jnp.float32),
        pltpu.SemaphoreType.DMA((NG,)),
        pltpu.SemaphoreType.DMA((NO,)),
    ],
    compiler_params=pltpu.CompilerParams(
        use_tc_tiling_on_sc=False, needs_layout_passes=False
    ),
)
def _gather_kernel(table_hbm, idx_hbm, out_hbm, idx_v, gbuf, obuf, gsem, osem):
    wid = lax.axis_index("s") * NC + lax.axis_index("c")
    t0 = wid * IPW
    pltpu.sync_copy(idx_hbm.at[pl.ds(t0, IPW)], idx_v)

    lane = lax.broadcasted_iota(jnp.int32, (16,), 0)
    lane16 = lane + 16

    def fire(k, g):
        pltpu.async_copy(table_hbm.at[idx_v.at[k]], gbuf.at[g], gsem.at[g])

    def gwait(g):
        pltpu.make_async_copy(
            table_hbm.at[pl.ds(0, L)], gbuf.at[g], gsem.at[g]
        ).wait()

    def owait(o):
        for ti in range(FT):
            pltpu.make_async_copy(
                obuf.at[o, pl.ds(ti * 8, 8), pl.ds(0, L)],
                out_hbm.at[0, ti, 0],
                osem.at[o],
            ).wait()

    for g in range(NG):
        fire(g, g)

    def item(k0):
        for u in range(NG):
            g = u % NG
            o = u % NO
            k = k0 + u
            t = t0 + k
            s = t // L
            tj = lax.rem(t, L)

            gwait(g)

            @pl.when(k >= NO)
            def _():
                owait(o)

            def rows(c):
                cvec = jnp.full((16,), c, jnp.int32)
                v0 = gbuf[g, c, pl.ds(0, 16)]
                v1 = gbuf[g, c, pl.ds(16, 16)]
                plsc.store_scatter(obuf.at[o], [lane, cvec], v0)
                plsc.store_scatter(obuf.at[o], [lane16, cvec], v1)

            pl.loop(0, L, unroll=8)(rows)

            for ti in range(FT):
                pltpu.async_copy(
                    obuf.at[o, pl.ds(ti * 8, 8), pl.ds(0, L)],
                    out_hbm.at[s, ti, tj],
                    osem.at[o],
                )

            @pl.when(k + NG < IPW)
            def _():
                fire(k + NG, g)

    pl.loop(0, IPW, step=NG)(item)

    for o in range(NO):
        owait(o)


@jax.jit
def kernel(inputs, embeddings):
    idx2 = inputs.T.reshape(NITEMS, L).astype(jnp.int32)
    out5 = _gather_kernel(embeddings, idx2)
    return out5.transpose(2, 4, 0, 1, 3).reshape(B, S, D)
